# baseline (speedup 1.0000x reference)
import functools

import jax
import jax.numpy as jnp
from jax import lax
from jax.experimental import pallas as pl
from jax.experimental.pallas import tpu as pltpu
from jax.experimental.pallas import tpu_sc as plsc

_B, _H, _W = 4, 224, 224
_HW = _H * _W
_T = 224
_C2 = 256
_NW = 32
_CP = _C2 // _NW
_CHUNK = 3136
_NCH = _HW // _CHUNK
_NT = _B * _NCH
_VPC = _CHUNK // 16

_mesh = plsc.VectorSubcoreMesh(core_axis_name="c", subcore_axis_name="s")


@functools.partial(
    pl.kernel,
    mesh=_mesh,
    out_type=jax.ShapeDtypeStruct((_B * _C2 * _HW,), jnp.float32),
    scratch_types=[
        pltpu.VMEM((_CP * _T,), jnp.float32),
        pltpu.VMEM((_CHUNK,), jnp.int32),
        pltpu.VMEM((_CP * _CHUNK,), jnp.float32),
    ],
    compiler_params=pltpu.CompilerParams(needs_layout_passes=False),
)
def _sc_kernel(tcat_hbm, ij_hbm, out_hbm, tbl_v, idx_v, outb_v):
    wid = lax.axis_index("s") * 2 + lax.axis_index("c")
    c0 = wid * _CP
    tsel = wid // (_NW // 2)
    pltpu.sync_copy(tcat_hbm.at[pl.ds(c0 * _T, _CP * _T)], tbl_v)

    def chunk_body(t, carry):
        b = t // _NCH
        off = (t % _NCH) * _CHUNK
        src = tsel * (_B * _HW) + b * _HW + off
        pltpu.sync_copy(ij_hbm.at[pl.ds(src, _CHUNK)], idx_v)

        def vec_body(v, c2):
            pos = pl.multiple_of(v * 16, 16)
            idxv = idx_v[pl.ds(pos, 16)]
            for cc in range(_CP):
                vals = plsc.load_gather(tbl_v, [idxv + cc * _T])
                outb_v[pl.ds(cc * _CHUNK + pos, 16)] = vals
            return c2

        lax.fori_loop(0, _VPC, vec_body, 0)

        dstbase = (b * _C2 + c0) * _HW + off
        for cc in range(_CP):
            pltpu.sync_copy(
                outb_v.at[pl.ds(cc * _CHUNK, _CHUNK)],
                out_hbm.at[pl.ds(dstbase + cc * _HW, _CHUNK)])
        return carry

    lax.fori_loop(0, _NT, chunk_body, 0)


def kernel(i, j, table_i, table_j):
    tcat = jnp.concatenate([table_i.T, table_j.T], axis=0).reshape(-1)
    ij = jnp.stack([i.reshape(_B, _HW), j.reshape(_B, _HW)]
                   ).reshape(-1).astype(jnp.int32)
    out = _sc_kernel(tcat, ij)
    return out.reshape(_B, _C2, _H, _W)
